# baseline (speedup 1.0000x reference)
import dataclasses
import functools

import jax
import jax.numpy as jnp
from jax import lax
from jax.experimental import pallas as pl
from jax.experimental.pallas import tpu as pltpu
from jax.experimental.pallas import tpu_sc as plsc

BS, FL, BAG, D = 1024, 100, 20, 32
NB = FL * BAG
RB = 128
NC, NS = 2, 16
NW = NC * NS
BPW = BS // NW
RBLK = 8
NBLK = BPW // RBLK
UNROLL = 8
MAXN = 1.0
EPS = 1e-7


def _sc_hist(addr_flat):
    mesh = plsc.VectorSubcoreMesh(core_axis_name="c", subcore_axis_name="s")
    cp = pltpu.CompilerParams()
    if "needs_layout_passes" in pltpu.CompilerParams.__dataclass_fields__:
        cp = dataclasses.replace(cp, needs_layout_passes=False)

    @functools.partial(
        pl.kernel,
        out_type=jax.ShapeDtypeStruct((BS * RB,), jnp.float32),
        mesh=mesh,
        compiler_params=cp,
        scratch_types=[
            pltpu.VMEM((RBLK * NB,), jnp.int32),
            pltpu.VMEM((RBLK * NB,), jnp.int32),
            pltpu.VMEM((RBLK * RB,), jnp.float32),
            pltpu.VMEM((RBLK * RB,), jnp.float32),
            pltpu.SemaphoreType.DMA,
            pltpu.SemaphoreType.DMA,
            pltpu.SemaphoreType.DMA,
            pltpu.SemaphoreType.DMA,
        ],
    )
    def k(addr_hbm, out_hbm, b0, b1, a0, a1, si0, si1, so0, so1):
        wid = lax.axis_index("s") * NC + lax.axis_index("c")
        row0 = wid * BPW
        bufs, accs = (b0, b1), (a0, a1)
        sis, sos = (si0, si1), (so0, so1)

        ones = jnp.ones((16,), jnp.float32)
        zero = jnp.zeros((16,), jnp.float32)

        in_cp = [None, None]
        out_cp = [None, None]
        in_cp[0] = pltpu.async_copy(
            addr_hbm.at[pl.ds(row0 * NB, RBLK * NB)], b0, si0)

        for blk in range(NBLK):
            p = blk % 2
            buf, acc = bufs[p], accs[p]
            if blk + 1 < NBLK:
                q = (blk + 1) % 2
                in_cp[q] = pltpu.async_copy(
                    addr_hbm.at[pl.ds((row0 + (blk + 1) * RBLK) * NB,
                                      RBLK * NB)],
                    bufs[q], sis[q])
            if out_cp[p] is not None:
                out_cp[p].wait()

            @pl.loop(0, RBLK * RB, step=64)
            def _(i):
                acc[pl.ds(i, 16)] = zero
                acc[pl.ds(i + 16, 16)] = zero
                acc[pl.ds(i + 32, 16)] = zero
                acc[pl.ds(i + 48, 16)] = zero

            in_cp[p].wait()

            @pl.loop(0, RBLK * NB, step=16 * UNROLL)
            def _(c):
                for u in range(UNROLL):
                    a = buf[pl.ds(c + 16 * u, 16)]
                    plsc.addupdate_scatter(acc, [a], ones)

            out_cp[p] = pltpu.async_copy(
                acc, out_hbm.at[pl.ds((row0 + blk * RBLK) * RB, RBLK * RB)],
                sos[p])

        out_cp[0].wait()
        out_cp[1].wait()

    return k(addr_flat)


def _tc_combine(W, mask_pad, pt_pad, vt_pad):
    def body(w_ref, m_ref, pt_ref, vt_ref, o_ref):
        def renorm(x):
            n = jnp.sqrt(jnp.sum(x * x, axis=-1, keepdims=True))
            return x * jnp.minimum(1.0, MAXN / jnp.maximum(n, EPS))

        vt = renorm(vt_ref[...])
        pt = renorm(pt_ref[...])
        acc = jnp.dot(w_ref[...], vt, preferred_element_type=jnp.float32,
                      precision=lax.Precision.HIGHEST)
        acc = acc + jnp.dot(m_ref[...], pt, preferred_element_type=jnp.float32,
                            precision=lax.Precision.HIGHEST)
        o_ref[...] = acc * (1.0 / FL)

    return pl.pallas_call(
        body,
        out_shape=jax.ShapeDtypeStruct((BS, D), jnp.float32),
    )(W, mask_pad, pt_pad, vt_pad)


def kernel(value_bin_ind, bin_mask, pos_table, val_table):
    idx = value_bin_ind.astype(jnp.int32)
    masked = jnp.where((bin_mask > 0)[:, :, None], idx, FL)
    row_off = (jnp.arange(BS, dtype=jnp.int32) % RBLK) * RB
    addr = masked.reshape(BS, NB) + row_off[:, None]
    W = _sc_hist(addr.reshape(-1)).reshape(BS, RB)
    mask_pad = jnp.pad(bin_mask.astype(jnp.float32), ((0, 0), (0, RB - FL)))
    pt_pad = jnp.pad(pos_table, ((0, RB - FL), (0, 0)))
    vt_pad = jnp.pad(val_table, ((0, RB - FL), (0, 0)))
    return _tc_combine(W, mask_pad, pt_pad, vt_pad)


# speedup vs baseline: 1.2329x; 1.2329x over previous
import dataclasses
import functools

import jax
import jax.numpy as jnp
from jax import lax
from jax.experimental import pallas as pl
from jax.experimental.pallas import tpu as pltpu
from jax.experimental.pallas import tpu_sc as plsc

BS, FL, BAG, D = 1024, 100, 20, 32
NB = FL * BAG
RB = 128
NC, NS = 2, 16
NW = NC * NS
BPW = BS // NW
RBLK = 8
NBLK = BPW // RBLK
UNROLL = 8
MAXN = 1.0
EPS = 1e-7


def _sc_hist(addr_flat):
    mesh = plsc.VectorSubcoreMesh(core_axis_name="c", subcore_axis_name="s")
    cp = pltpu.CompilerParams()
    if "needs_layout_passes" in pltpu.CompilerParams.__dataclass_fields__:
        cp = dataclasses.replace(cp, needs_layout_passes=False)

    @functools.partial(
        pl.kernel,
        out_type=jax.ShapeDtypeStruct((BS * RB,), jnp.float32),
        mesh=mesh,
        compiler_params=cp,
        scratch_types=[
            pltpu.VMEM((RBLK * NB,), jnp.int32),
            pltpu.VMEM((RBLK * NB,), jnp.int32),
            pltpu.VMEM((RBLK * RB,), jnp.float32),
            pltpu.VMEM((RBLK * RB,), jnp.float32),
            pltpu.SemaphoreType.DMA,
            pltpu.SemaphoreType.DMA,
            pltpu.SemaphoreType.DMA,
            pltpu.SemaphoreType.DMA,
        ],
    )
    def k(addr_hbm, out_hbm, b0, b1, a0, a1, si0, si1, so0, so1):
        wid = lax.axis_index("s") * NC + lax.axis_index("c")
        row0 = wid * BPW
        bufs, accs = (b0, b1), (a0, a1)
        sis, sos = (si0, si1), (so0, so1)

        ones = jnp.ones((16,), jnp.float32)
        zero = jnp.zeros((16,), jnp.float32)

        in_cp = [None, None]
        out_cp = [None, None]
        in_cp[0] = pltpu.async_copy(
            addr_hbm.at[pl.ds(row0 * NB, RBLK * NB)], b0, si0)

        for blk in range(NBLK):
            p = blk % 2
            buf, acc = bufs[p], accs[p]
            if blk + 1 < NBLK:
                q = (blk + 1) % 2
                in_cp[q] = pltpu.async_copy(
                    addr_hbm.at[pl.ds((row0 + (blk + 1) * RBLK) * NB,
                                      RBLK * NB)],
                    bufs[q], sis[q])
            if out_cp[p] is not None:
                out_cp[p].wait()

            @pl.loop(0, RBLK * RB, step=64)
            def _(i):
                acc[pl.ds(i, 16)] = zero
                acc[pl.ds(i + 16, 16)] = zero
                acc[pl.ds(i + 32, 16)] = zero
                acc[pl.ds(i + 48, 16)] = zero

            in_cp[p].wait()

            @pl.loop(0, RBLK * NB, step=16 * UNROLL)
            def _(c):
                for u in range(UNROLL):
                    a = buf[pl.ds(c + 16 * u, 16)]
                    plsc.addupdate_scatter(acc, [a], ones)

            out_cp[p] = pltpu.async_copy(
                acc, out_hbm.at[pl.ds((row0 + blk * RBLK) * RB, RBLK * RB)],
                sos[p])

        out_cp[0].wait()
        out_cp[1].wait()

    return k(addr_flat)


def _tc_combine(W, mask_pad, pt_pad, vt_pad):
    def body(w_ref, m_ref, pt_ref, vt_ref, o_ref):
        def renorm(x):
            n = jnp.sqrt(jnp.sum(x * x, axis=-1, keepdims=True))
            return x * jnp.minimum(1.0, MAXN / jnp.maximum(n, EPS))

        vt = renorm(vt_ref[...])
        pt = renorm(pt_ref[...])
        acc = jnp.dot(w_ref[...], vt, preferred_element_type=jnp.float32,
                      precision=lax.Precision.HIGHEST)
        acc = acc + jnp.dot(m_ref[...], pt, preferred_element_type=jnp.float32,
                            precision=lax.Precision.HIGHEST)
        o_ref[...] = acc * (1.0 / FL)

    return pl.pallas_call(
        body,
        out_shape=jax.ShapeDtypeStruct((BS, D), jnp.float32),
    )(W, mask_pad, pt_pad, vt_pad)


def kernel(value_bin_ind, bin_mask, pos_table, val_table):
    idx = value_bin_ind.astype(jnp.int32)
    dead = FL + (jnp.arange(NB, dtype=jnp.int32) % (RB - FL)).reshape(FL, BAG)
    masked = jnp.where((bin_mask > 0)[:, :, None], idx, dead[None])
    row_off = (jnp.arange(BS, dtype=jnp.int32) % RBLK) * RB
    addr = masked.reshape(BS, NB) + row_off[:, None]
    W = _sc_hist(addr.reshape(-1)).reshape(BS, RB)
    mask_pad = jnp.pad(bin_mask.astype(jnp.float32), ((0, 0), (0, RB - FL)))
    pt_pad = jnp.pad(pos_table, ((0, RB - FL), (0, 0)))
    vt_pad = jnp.pad(val_table, ((0, RB - FL), (0, 0)))
    return _tc_combine(W, mask_pad, pt_pad, vt_pad)


# speedup vs baseline: 1.9247x; 1.5611x over previous
import dataclasses
import functools

import jax
import jax.numpy as jnp
from jax import lax
from jax.experimental import pallas as pl
from jax.experimental.pallas import tpu as pltpu
from jax.experimental.pallas import tpu_sc as plsc

BS, FL, BAG, D = 1024, 100, 20, 32
NB = FL * BAG
RB = 128
NC, NS = 2, 16
NW = NC * NS
BPW = BS // NW
RBLK = 16
NBLK = BPW // RBLK
UNROLL = 4
MAXN = 1.0
EPS = 1e-7


def _sc_hist(addr_flat):
    mesh = plsc.VectorSubcoreMesh(core_axis_name="c", subcore_axis_name="s")
    cp = pltpu.CompilerParams()
    if "needs_layout_passes" in pltpu.CompilerParams.__dataclass_fields__:
        cp = dataclasses.replace(cp, needs_layout_passes=False)

    @functools.partial(
        pl.kernel,
        out_type=jax.ShapeDtypeStruct((BS * RB,), jnp.float32),
        mesh=mesh,
        compiler_params=cp,
        scratch_types=[
            pltpu.VMEM((RBLK * NB // 2,), jnp.int32),
            pltpu.VMEM((RBLK * NB // 2,), jnp.int32),
            pltpu.VMEM((RBLK * RB,), jnp.float32),
            pltpu.VMEM((RBLK * RB,), jnp.float32),
            pltpu.SemaphoreType.DMA,
            pltpu.SemaphoreType.DMA,
            pltpu.SemaphoreType.DMA,
            pltpu.SemaphoreType.DMA,
        ],
    )
    def k(addr_hbm, out_hbm, b0, b1, a0, a1, si0, si1, so0, so1):
        wid = lax.axis_index("s") * NC + lax.axis_index("c")
        row0 = wid * BPW
        bufs, accs = (b0, b1), (a0, a1)
        sis, sos = (si0, si1), (so0, so1)

        ones = jnp.ones((16,), jnp.float32)
        zero = jnp.zeros((16,), jnp.float32)

        in_cp = [None, None]
        out_cp = [None, None]
        in_cp[0] = pltpu.async_copy(
            addr_hbm.at[pl.ds(row0 * (NB // 2), RBLK * NB // 2)], b0, si0)

        for blk in range(NBLK):
            p = blk % 2
            buf, acc = bufs[p], accs[p]
            if blk + 1 < NBLK:
                q = (blk + 1) % 2
                in_cp[q] = pltpu.async_copy(
                    addr_hbm.at[pl.ds((row0 + (blk + 1) * RBLK) * (NB // 2),
                                      RBLK * NB // 2)],
                    bufs[q], sis[q])
            if out_cp[p] is not None:
                out_cp[p].wait()

            @pl.loop(0, RBLK * RB, step=64)
            def _(i):
                acc[pl.ds(i, 16)] = zero
                acc[pl.ds(i + 16, 16)] = zero
                acc[pl.ds(i + 32, 16)] = zero
                acc[pl.ds(i + 48, 16)] = zero

            in_cp[p].wait()

            @pl.loop(0, RBLK * NB // 2, step=16 * UNROLL)
            def _(c):
                for u in range(UNROLL):
                    packed = buf[pl.ds(c + 16 * u, 16)]
                    a0 = packed & 0xFFFF
                    a1 = lax.shift_right_logical(packed, 16)
                    plsc.addupdate_scatter(acc, [a0], ones)
                    plsc.addupdate_scatter(acc, [a1], ones)

            out_cp[p] = pltpu.async_copy(
                acc, out_hbm.at[pl.ds((row0 + blk * RBLK) * RB, RBLK * RB)],
                sos[p])

        out_cp[0].wait()
        out_cp[1].wait()

    return k(addr_flat)


def _tc_combine(W, mask_pad, pt_pad, vt_pad):
    def body(w_ref, m_ref, pt_ref, vt_ref, o_ref):
        def renorm(x):
            n = jnp.sqrt(jnp.sum(x * x, axis=-1, keepdims=True))
            return x * jnp.minimum(1.0, MAXN / jnp.maximum(n, EPS))

        vt = renorm(vt_ref[...])
        pt = renorm(pt_ref[...])
        acc = jnp.dot(w_ref[...], vt, preferred_element_type=jnp.float32,
                      precision=lax.Precision.HIGHEST)
        acc = acc + jnp.dot(m_ref[...], pt, preferred_element_type=jnp.float32,
                            precision=lax.Precision.HIGHEST)
        o_ref[...] = acc * (1.0 / FL)

    return pl.pallas_call(
        body,
        out_shape=jax.ShapeDtypeStruct((BS, D), jnp.float32),
    )(W, mask_pad, pt_pad, vt_pad)


def kernel(value_bin_ind, bin_mask, pos_table, val_table):
    idx = value_bin_ind.astype(jnp.int32)
    dead = FL + (jnp.arange(NB, dtype=jnp.int32) % 27).reshape(FL, BAG)
    masked = jnp.where((bin_mask > 0)[:, :, None], idx, dead[None])
    row_off = (jnp.arange(BS, dtype=jnp.int32) % RBLK) * RB
    addr = masked.reshape(BS, NB) + row_off[:, None]
    pairs = addr.reshape(BS, NB // 2, 2)
    packed = pairs[:, :, 0] | (pairs[:, :, 1] << 16)
    W = _sc_hist(packed.reshape(-1)).reshape(BS, RB)
    mask_pad = jnp.pad(bin_mask.astype(jnp.float32), ((0, 0), (0, RB - FL)))
    pt_pad = jnp.pad(pos_table, ((0, RB - FL), (0, 0)))
    vt_pad = jnp.pad(val_table, ((0, RB - FL), (0, 0)))
    return _tc_combine(W, mask_pad, pt_pad, vt_pad)


# speedup vs baseline: 1.9379x; 1.0068x over previous
import dataclasses
import functools

import jax
import jax.numpy as jnp
from jax import lax
from jax.experimental import pallas as pl
from jax.experimental.pallas import tpu as pltpu
from jax.experimental.pallas import tpu_sc as plsc

BS, FL, BAG, D = 1024, 100, 20, 32
NB = FL * BAG
RB = 128
NC, NS = 2, 16
NW = NC * NS
BPW = BS // NW
RBLK = 16
WPR = 1024
NBLK = BPW // RBLK
UNROLL = 4
MAXN = 1.0
EPS = 1e-7


def _sc_hist(addr_flat):
    mesh = plsc.VectorSubcoreMesh(core_axis_name="c", subcore_axis_name="s")
    cp = pltpu.CompilerParams(use_tc_tiling_on_sc=True)
    if "needs_layout_passes" in pltpu.CompilerParams.__dataclass_fields__:
        cp = dataclasses.replace(cp, needs_layout_passes=False)

    @functools.partial(
        pl.kernel,
        out_type=jax.ShapeDtypeStruct((BS * RB,), jnp.float32),
        mesh=mesh,
        compiler_params=cp,
        scratch_types=[
            pltpu.VMEM((RBLK, WPR), jnp.int32),
            pltpu.VMEM((RBLK, WPR), jnp.int32),
            pltpu.VMEM((RBLK * RB,), jnp.float32),
            pltpu.VMEM((RBLK * RB,), jnp.float32),
            pltpu.SemaphoreType.DMA,
            pltpu.SemaphoreType.DMA,
            pltpu.SemaphoreType.DMA,
            pltpu.SemaphoreType.DMA,
        ],
    )
    def k(addr_hbm, out_hbm, b0, b1, a0, a1, si0, si1, so0, so1):
        wid = lax.axis_index("s") * NC + lax.axis_index("c")
        row0 = wid * BPW
        bufs, accs = (b0, b1), (a0, a1)
        sis, sos = (si0, si1), (so0, so1)

        ones = jnp.ones((16,), jnp.float32)
        zero = jnp.zeros((16,), jnp.float32)

        in_cp = [None, None]
        out_cp = [None, None]
        in_cp[0] = pltpu.async_copy(
            addr_hbm.at[pl.ds(row0, RBLK)], b0, si0)

        for blk in range(NBLK):
            p = blk % 2
            buf, acc = bufs[p], accs[p]
            if blk + 1 < NBLK:
                q = (blk + 1) % 2
                in_cp[q] = pltpu.async_copy(
                    addr_hbm.at[pl.ds(row0 + (blk + 1) * RBLK, RBLK)],
                    bufs[q], sis[q])
            if out_cp[p] is not None:
                out_cp[p].wait()

            @pl.loop(0, RBLK * RB, step=64)
            def _(i):
                acc[pl.ds(i, 16)] = zero
                acc[pl.ds(i + 16, 16)] = zero
                acc[pl.ds(i + 32, 16)] = zero
                acc[pl.ds(i + 48, 16)] = zero

            in_cp[p].wait()

            @pl.loop(0, RBLK)
            def _(r):
                @pl.loop(0, WPR, step=16 * UNROLL)
                def _(c):
                    for u in range(UNROLL):
                        packed = buf[r, pl.ds(c + 16 * u, 16)]
                        a0 = packed & 0xFFFF
                        a1 = lax.shift_right_logical(packed, 16)
                        plsc.addupdate_scatter(acc, [a0], ones)
                        plsc.addupdate_scatter(acc, [a1], ones)

            out_cp[p] = pltpu.async_copy(
                acc, out_hbm.at[pl.ds((row0 + blk * RBLK) * RB, RBLK * RB)],
                sos[p])

        out_cp[0].wait()
        out_cp[1].wait()

    return k(addr_flat)


def _tc_combine(W, mask_pad, pt_pad, vt_pad):
    def body(w_ref, m_ref, pt_ref, vt_ref, o_ref):
        def renorm(x):
            n = jnp.sqrt(jnp.sum(x * x, axis=-1, keepdims=True))
            return x * jnp.minimum(1.0, MAXN / jnp.maximum(n, EPS))

        vt = renorm(vt_ref[...])
        pt = renorm(pt_ref[...])
        acc = jnp.dot(w_ref[...], vt, preferred_element_type=jnp.float32,
                      precision=lax.Precision.HIGHEST)
        acc = acc + jnp.dot(m_ref[...], pt, preferred_element_type=jnp.float32,
                            precision=lax.Precision.HIGHEST)
        o_ref[...] = acc * (1.0 / FL)

    return pl.pallas_call(
        body,
        out_shape=jax.ShapeDtypeStruct((BS, D), jnp.float32),
    )(W, mask_pad, pt_pad, vt_pad)


def kernel(value_bin_ind, bin_mask, pos_table, val_table):
    idx = value_bin_ind.astype(jnp.int32)
    dead = FL + (jnp.arange(NB, dtype=jnp.int32) % 27).reshape(FL, BAG)
    masked = jnp.where((bin_mask > 0)[:, :, None], idx, dead[None])
    row_off = (jnp.arange(BS, dtype=jnp.int32) % RBLK) * RB
    addr = masked.reshape(BS, NB) + row_off[:, None]
    pairs = addr.reshape(BS, NB // 2, 2)
    packed = pairs[:, :, 0] | (pairs[:, :, 1] << 16)
    pj = jnp.arange(WPR - NB // 2, dtype=jnp.int32)
    padpat = (FL + (2 * pj) % 27) | ((FL + (2 * pj + 1) % 27) << 16)
    packed = jnp.concatenate(
        [packed, jnp.broadcast_to(padpat[None], (BS, WPR - NB // 2))], axis=1)
    W = _sc_hist(packed).reshape(BS, RB)
    mask_pad = jnp.pad(bin_mask.astype(jnp.float32), ((0, 0), (0, RB - FL)))
    pt_pad = jnp.pad(pos_table, ((0, RB - FL), (0, 0)))
    vt_pad = jnp.pad(val_table, ((0, RB - FL), (0, 0)))
    return _tc_combine(W, mask_pad, pt_pad, vt_pad)
